# baseline (speedup 1.0000x reference)
import functools

import jax
import jax.numpy as jnp
from jax import lax
from jax.experimental import pallas as pl
from jax.experimental.pallas import tpu as pltpu
from jax.experimental.pallas import tpu_sc as plsc

B, NB, PL, H, F = 16, 64, 24, 1024, 4096
M = B * NB
NC, NS, L = 2, 16, 16
NW = NC * NS
ROWS = M // NW
CH = H // L

BMV = 256
BML = 256


NBUF = 4
UNROLL = 4


def _sc_pool_body(lang_hbm, out_hbm, buf, orow, isem0, isem1, isem2, isem3,
                  osem0, osem1):
    wid = lax.axis_index("s") * NC + lax.axis_index("c")
    base = wid * ROWS
    isems = (isem0, isem1, isem2, isem3)
    osems = (osem0, osem1)

    for b in range(NBUF):
        pltpu.async_copy(lang_hbm.at[base + b], buf.at[b], isems[b])

    def row_group(i, carry):
        r0 = NBUF * i
        for b in range(NBUF):
            r = r0 + b
            ob = b % 2
            pltpu.make_async_copy(lang_hbm.at[base + r], buf.at[b],
                                  isems[b]).wait()

            @pl.when(r >= 2)
            def _():
                pltpu.make_async_copy(orow.at[ob],
                                      out_hbm.at[base + r - 2],
                                      osems[ob]).wait()

            def chunk(c, carry2):
                for u in range(UNROLL):
                    off = c * (UNROLL * L) + u * L
                    acc = buf[b, 0, pl.ds(off, L)]
                    for t in range(1, PL):
                        acc = acc + buf[b, t, pl.ds(off, L)]
                    orow[ob, pl.ds(off, L)] = acc
                return carry2

            lax.fori_loop(0, CH // UNROLL, chunk, 0)
            pltpu.async_copy(orow.at[ob], out_hbm.at[base + r], osems[ob])

            @pl.when(r + NBUF < ROWS)
            def _():
                pltpu.async_copy(lang_hbm.at[base + r + NBUF], buf.at[b],
                                 isems[b])

        return carry

    lax.fori_loop(0, ROWS // NBUF, row_group, 0)
    pltpu.make_async_copy(orow.at[0], out_hbm.at[base + ROWS - 2],
                          osems[0]).wait()
    pltpu.make_async_copy(orow.at[1], out_hbm.at[base + ROWS - 1],
                          osems[1]).wait()


_sc_pool = functools.partial(
    pl.kernel,
    out_type=jax.ShapeDtypeStruct((M, H), jnp.float32),
    mesh=plsc.VectorSubcoreMesh(core_axis_name="c", subcore_axis_name="s",
                                num_cores=NC, num_subcores=NS),
    scratch_types=[
        pltpu.VMEM((NBUF, PL, H), jnp.float32),
        pltpu.VMEM((2, H), jnp.float32),
        pltpu.SemaphoreType.DMA,
        pltpu.SemaphoreType.DMA,
        pltpu.SemaphoreType.DMA,
        pltpu.SemaphoreType.DMA,
        pltpu.SemaphoreType.DMA,
        pltpu.SemaphoreType.DMA,
    ],
    cost_estimate=pl.CostEstimate(
        flops=M * PL * H,
        bytes_accessed=M * PL * H * 4 + M * H * 4,
        transcendentals=0,
    ),
)(_sc_pool_body)


def _vis_body(vis_ref, wv_ref, bv_ref, out_ref):
    out_ref[...] = (
        lax.dot_general(vis_ref[...], wv_ref[...], (((1,), (1,)), ((), ())),
                        preferred_element_type=jnp.float32)
        + bv_ref[...]
    )


def _lang_body(pooled_ref, invlen_ref, wl_ref, bl_ref, out_ref):
    scaled = pooled_ref[...] * invlen_ref[...]
    out_ref[...] = (
        lax.dot_general(scaled, wl_ref[...], (((1,), (1,)), ((), ())),
                        preferred_element_type=jnp.float32)
        + bl_ref[...]
    )


@functools.partial(jax.jit, static_argnames=())
def kernel(vision, language, phrase_lengths, Wv, bv, Wl, bl):
    vis = vision.reshape(M, F)
    lang = language.reshape(M, PL, H)
    inv_len = (1.0 / phrase_lengths.astype(jnp.float32)).reshape(M, 1)

    pooled = _sc_pool(lang)

    vmap = pl.pallas_call(
        _vis_body,
        grid=(M // BMV,),
        in_specs=[
            pl.BlockSpec((BMV, F), lambda i: (i, 0)),
            pl.BlockSpec((H, F), lambda i: (0, 0)),
            pl.BlockSpec((1, H), lambda i: (0, 0)),
        ],
        out_specs=pl.BlockSpec((BMV, H), lambda i: (i, 0)),
        out_shape=jax.ShapeDtypeStruct((M, H), jnp.float32),
    )(vis, Wv, bv.reshape(1, H))

    lmap = pl.pallas_call(
        _lang_body,
        grid=(M // BML,),
        in_specs=[
            pl.BlockSpec((BML, H), lambda i: (i, 0)),
            pl.BlockSpec((BML, 1), lambda i: (i, 0)),
            pl.BlockSpec((H, H), lambda i: (0, 0)),
            pl.BlockSpec((1, H), lambda i: (0, 0)),
        ],
        out_specs=pl.BlockSpec((BML, H), lambda i: (i, 0)),
        out_shape=jax.ShapeDtypeStruct((M, H), jnp.float32),
    )(pooled, inv_len, Wl, bl.reshape(1, H))

    return (lmap.reshape(B, NB, H), vmap.reshape(B, NB, H))


# speedup vs baseline: 1.7976x; 1.7976x over previous
import functools

import jax
import jax.numpy as jnp
from jax import lax
from jax.experimental import pallas as pl
from jax.experimental.pallas import tpu as pltpu

B, NB, PL, H, F = 16, 64, 24, 1024, 4096
M = B * NB
BM = 128
NSTEP = M // BM
LAG = 2
NRING = LAG + 1


def _body(vis_ref, lang_ref, invlen_ref, wv_hbm, bv_hbm, wl_hbm, bl_hbm,
          lmap_ref, vmap_ref,
          wv_v, wl_v, bv_v, bl_v, ring, sem_wv, sem_wl, sem_bv, sem_bl):
    i = pl.program_id(0)

    @pl.when(i == 0)
    def _():
        pltpu.async_copy(wl_hbm, wl_v, sem_wl)
        pltpu.async_copy(bv_hbm, bv_v, sem_bv)
        pltpu.async_copy(bl_hbm, bl_v, sem_bl)
        pltpu.async_copy(wv_hbm, wv_v, sem_wv)

    @pl.when(i < NSTEP)
    def _():
        slot = lax.rem(i, NRING)
        pooled = jnp.sum(lang_ref[...], axis=1) * invlen_ref[...]
        ring[slot] = pooled

    @pl.when(i == LAG)
    def _():
        pltpu.make_async_copy(wl_hbm, wl_v, sem_wl).wait()
        pltpu.make_async_copy(bv_hbm, bv_v, sem_bv).wait()
        pltpu.make_async_copy(bl_hbm, bl_v, sem_bl).wait()
        pltpu.make_async_copy(wv_hbm, wv_v, sem_wv).wait()

    @pl.when(i >= LAG)
    def _():
        slot = lax.rem(i - LAG, NRING)
        lmap_ref[...] = (
            lax.dot_general(ring[slot], wl_v[...], (((1,), (1,)), ((), ())),
                            preferred_element_type=jnp.float32)
            + bl_v[...]
        )
        vmap_ref[...] = (
            lax.dot_general(vis_ref[...], wv_v[...], (((1,), (1,)), ((), ())),
                            preferred_element_type=jnp.float32)
            + bv_v[...]
        )


@functools.partial(jax.jit, static_argnames=())
def kernel(vision, language, phrase_lengths, Wv, bv, Wl, bl):
    vis = vision.reshape(M, F)
    lang = language.reshape(M, PL, H)
    inv_len = (1.0 / phrase_lengths.astype(jnp.float32)).reshape(M, 1)

    def fwd(i):
        return (jnp.minimum(i, NSTEP - 1),)

    def lagged(i):
        return (jnp.maximum(i - LAG, 0),)

    lmap, vmap = pl.pallas_call(
        _body,
        grid=(NSTEP + LAG,),
        in_specs=[
            pl.BlockSpec((BM, F), lambda i: (jnp.maximum(i - LAG, 0), 0)),
            pl.BlockSpec((BM, PL, H), lambda i: (jnp.minimum(i, NSTEP - 1), 0, 0)),
            pl.BlockSpec((BM, 1), lambda i: (jnp.minimum(i, NSTEP - 1), 0)),
            pl.BlockSpec(memory_space=pl.ANY),
            pl.BlockSpec(memory_space=pl.ANY),
            pl.BlockSpec(memory_space=pl.ANY),
            pl.BlockSpec(memory_space=pl.ANY),
        ],
        out_specs=[
            pl.BlockSpec((BM, H), lambda i: (jnp.maximum(i - LAG, 0), 0)),
            pl.BlockSpec((BM, H), lambda i: (jnp.maximum(i - LAG, 0), 0)),
        ],
        out_shape=[
            jax.ShapeDtypeStruct((M, H), jnp.float32),
            jax.ShapeDtypeStruct((M, H), jnp.float32),
        ],
        scratch_shapes=[
            pltpu.VMEM((H, F), jnp.float32),
            pltpu.VMEM((H, H), jnp.float32),
            pltpu.VMEM((1, H), jnp.float32),
            pltpu.VMEM((1, H), jnp.float32),
            pltpu.VMEM((NRING, BM, H), jnp.float32),
            pltpu.SemaphoreType.DMA,
            pltpu.SemaphoreType.DMA,
            pltpu.SemaphoreType.DMA,
            pltpu.SemaphoreType.DMA,
        ],
    )(vis, lang, inv_len, Wv, bv.reshape(1, H), Wl, bl.reshape(1, H))

    return (lmap.reshape(B, NB, H), vmap.reshape(B, NB, H))
